# Initial kernel scaffold; baseline (speedup 1.0000x reference)
import jax
import jax.numpy as jnp
from jax.experimental import pallas as pl


def kernel(x, current_size):
    raise NotImplementedError("write your pallas kernel here")


# speedup vs baseline: 1.2579x; 1.2579x over previous
import functools

import jax
import jax.numpy as jnp
from jax import lax
from jax.experimental import pallas as pl
from jax.experimental.pallas import tpu as pltpu
from jax.experimental.pallas import tpu_sc as plsc

STEP_LEN = 8
MAX_SIZE = 400.0
MIN_SIZE = 1.0
DEFAULT_SIZE = 10.0

NC = 2
NS = 16
L = 16

B = 32
T = 4096
D = 256
ND = D // L
CH = 64


def _pool_params(cs_vec):
    ps = jnp.clip(cs_vec + DEFAULT_SIZE, MIN_SIZE, MAX_SIZE)
    trunc_i = ps.astype(jnp.int32)
    frac = ps - trunc_i.astype(jnp.float32)
    nb = (trunc_i + jnp.where(frac > 0.0, 1, 0))[0]
    return ps, nb


def _sc_body(x_hbm, cs_hbm, out_hbm, cs_v, buf, part_v, shared, sem):
    c = lax.axis_index("c")
    s = lax.axis_index("s")
    b1 = 2 * s + c
    b2 = 2 * (NS - 1 - s) + c

    pltpu.sync_copy(cs_hbm, cs_v)

    zeros = [jnp.zeros((L,), jnp.float32) for _ in range(2 * ND)]

    def do_range(b, lo, hi, ps, acc):
        nch = (hi - lo + CH - 1) // CH

        def chunk_body(i, acc):
            start = lo + i * CH
            pltpu.sync_copy(x_hbm.at[b, pl.ds(start, CH), :], buf)

            def row_body(r, acc):
                t = start + r
                tb = t // STEP_LEN
                w = jnp.clip(ps - tb.astype(jnp.float32), 0.0, 1.0)
                w = w * (t < hi).astype(jnp.float32)
                new_m = []
                new_s = []
                for j in range(ND):
                    v = buf[r, pl.ds(j * L, L)]
                    wv = w * v
                    new_m.append(acc[j] + wv)
                    new_s.append(acc[ND + j] + wv * v)
                return tuple(new_m + new_s)

            return lax.fori_loop(0, CH, row_body, acc, unroll=False)

        return lax.fori_loop(0, nch, chunk_body, acc, unroll=False)

    def read_cs(b):
        return plsc.load_gather(cs_v, [jnp.full((L,), b, jnp.int32)])

    cs1 = read_cs(b1)
    ps1, nb1 = _pool_params(cs1)
    h1 = STEP_LEN * ((nb1 + 1) // 2)
    acc = do_range(b1, 0, h1, ps1, tuple(zeros))
    for j in range(2 * ND):
        part_v[pl.ds(j * L, L)] = acc[j]
    pltpu.sync_copy(part_v, shared.at[s, 0])

    cs2 = read_cs(b2)
    ps2, nb2 = _pool_params(cs2)
    lo2 = STEP_LEN * ((nb2 + 1) // 2)
    hi2 = STEP_LEN * nb2
    acc = do_range(b2, lo2, hi2, ps2, tuple(zeros))
    for j in range(2 * ND):
        part_v[pl.ds(j * L, L)] = acc[j]
    pltpu.sync_copy(part_v, shared.at[NS - 1 - s, 1])

    plsc.subcore_barrier()

    pltpu.sync_copy(shared.at[s, 0], part_v)
    acc_a = [part_v[pl.ds(j * L, L)] for j in range(2 * ND)]
    pltpu.sync_copy(shared.at[s, 1], part_v)
    inv = 1.0 / (float(STEP_LEN) * ps1)
    for j in range(ND):
        m = (acc_a[j] + part_v[pl.ds(j * L, L)]) * inv
        sq = (acc_a[ND + j] + part_v[pl.ds((ND + j) * L, L)]) * inv
        part_v[pl.ds(j * L, L)] = m
        part_v[pl.ds((ND + j) * L, L)] = sq - m * m
    pltpu.sync_copy(part_v, out_hbm.at[b1])


@jax.jit
def _run(x, current_size):
    mesh = plsc.VectorSubcoreMesh(core_axis_name="c", subcore_axis_name="s")
    return pl.kernel(
        _sc_body,
        out_type=jax.ShapeDtypeStruct((B, 2 * D), jnp.float32),
        mesh=mesh,
        compiler_params=pltpu.CompilerParams(needs_layout_passes=False),
        scratch_types=[
            pltpu.VMEM((B,), jnp.float32),
            pltpu.VMEM((CH, D), jnp.float32),
            pltpu.VMEM((2 * D,), jnp.float32),
            pltpu.VMEM_SHARED((NS, 2, 2 * D), jnp.float32),
            pltpu.SemaphoreType.DMA,
        ],
    )(x, current_size)


def kernel(x, current_size):
    return _run(x, current_size)


# speedup vs baseline: 1.6455x; 1.3082x over previous
import functools

import jax
import jax.numpy as jnp
from jax import lax
from jax.experimental import pallas as pl
from jax.experimental.pallas import tpu as pltpu
from jax.experimental.pallas import tpu_sc as plsc

STEP_LEN = 8
MAX_SIZE = 400.0
MIN_SIZE = 1.0
DEFAULT_SIZE = 10.0

NC = 2
NS = 16
L = 16

B = 32
T = 4096
D = 256
ND = D // L
CH = 64


def _pool_params(cs_vec):
    ps = jnp.clip(cs_vec + DEFAULT_SIZE, MIN_SIZE, MAX_SIZE)
    trunc_v = ps.astype(jnp.int32)
    frac = ps - trunc_v.astype(jnp.float32)
    nb = (trunc_v + jnp.where(frac > 0.0, 1, 0))[0]
    return ps, trunc_v[0], nb


def _sc_body(x_hbm, cs_hbm, out_hbm, cs_v, buf0, buf1, acc_v, part_v, shared,
             sem0, sem1):
    c = lax.axis_index("c")
    s = lax.axis_index("s")
    b1 = 2 * s + c
    b2 = 2 * (NS - 1 - s) + c

    pltpu.sync_copy(cs_hbm, cs_v)

    zero = jnp.zeros((L,), jnp.float32)
    zeros = tuple(zero for _ in range(2 * ND))

    def read_cs(b):
        return plsc.load_gather(cs_v, [jnp.full((L,), b, jnp.int32)])

    def do_range(b, lo, hi, ps, trunc_s):
        for j in range(2 * ND):
            acc_v[pl.ds(j * L, L)] = zero
        nch = (hi - lo + CH - 1) // CH

        def flush(acc):
            for j in range(2 * ND):
                acc_v[pl.ds(j * L, L)] = acc_v[pl.ds(j * L, L)] + acc[j]

        def process(g, mybuf, mysem, other_buf, other_sem):
            pltpu.make_async_copy(
                x_hbm.at[b, pl.ds(0, CH), :], mybuf, mysem).wait()

            @pl.when(g + 1 < nch)
            def _prefetch():
                pltpu.async_copy(
                    x_hbm.at[b, pl.ds(lo + (g + 1) * CH, CH), :],
                    other_buf, other_sem)

            start = lo + g * CH
            tb_last = (start + CH - 1) // STEP_LEN
            fast = jnp.logical_and(start + CH <= hi, tb_last + 1 <= trunc_s)

            @pl.when(fast)
            def _fast():
                def row_body(r, acc):
                    new_m = []
                    new_s = []
                    for j in range(ND):
                        v = mybuf[r, pl.ds(j * L, L)]
                        new_m.append(acc[j] + v)
                        new_s.append(acc[ND + j] + v * v)
                    return tuple(new_m + new_s)

                flush(lax.fori_loop(0, CH, row_body, zeros, unroll=2))

            @pl.when(jnp.logical_not(fast))
            def _slow():
                def row_body(r, acc):
                    t = start + r
                    tb = t // STEP_LEN
                    w = jnp.clip(ps - tb.astype(jnp.float32), 0.0, 1.0)
                    w = w * (t < hi).astype(jnp.float32)
                    new_m = []
                    new_s = []
                    for j in range(ND):
                        v = mybuf[r, pl.ds(j * L, L)]
                        wv = w * v
                        new_m.append(acc[j] + wv)
                        new_s.append(acc[ND + j] + wv * v)
                    return tuple(new_m + new_s)

                flush(lax.fori_loop(0, CH, row_body, zeros, unroll=2))

        @pl.when(nch > 0)
        def _prime():
            pltpu.async_copy(x_hbm.at[b, pl.ds(lo, CH), :], buf0, sem0)

        def chunk_body(g, carry):
            @pl.when(g % 2 == 0)
            def _even():
                process(g, buf0, sem0, buf1, sem1)

            @pl.when(g % 2 == 1)
            def _odd():
                process(g, buf1, sem1, buf0, sem0)

            return carry

        lax.fori_loop(0, nch, chunk_body, 0, unroll=False)

    ps1, tr1, nb1 = _pool_params(read_cs(b1))
    h1 = STEP_LEN * ((nb1 + 1) // 2)
    do_range(b1, 0, h1, ps1, tr1)
    pltpu.sync_copy(acc_v, shared.at[s, 0])

    ps2, tr2, nb2 = _pool_params(read_cs(b2))
    lo2 = STEP_LEN * ((nb2 + 1) // 2)
    hi2 = STEP_LEN * nb2
    do_range(b2, lo2, hi2, ps2, tr2)
    pltpu.sync_copy(acc_v, shared.at[NS - 1 - s, 1])

    plsc.subcore_barrier()

    pltpu.sync_copy(shared.at[s, 0], acc_v)
    pltpu.sync_copy(shared.at[s, 1], part_v)
    inv = 1.0 / (float(STEP_LEN) * ps1)
    for j in range(ND):
        m = (acc_v[pl.ds(j * L, L)] + part_v[pl.ds(j * L, L)]) * inv
        sq = (acc_v[pl.ds((ND + j) * L, L)] + part_v[pl.ds((ND + j) * L, L)]) * inv
        part_v[pl.ds(j * L, L)] = m
        part_v[pl.ds((ND + j) * L, L)] = sq - m * m
    pltpu.sync_copy(part_v, out_hbm.at[b1])


@jax.jit
def _run(x, current_size):
    mesh = plsc.VectorSubcoreMesh(core_axis_name="c", subcore_axis_name="s")
    return pl.kernel(
        _sc_body,
        out_type=jax.ShapeDtypeStruct((B, 2 * D), jnp.float32),
        mesh=mesh,
        compiler_params=pltpu.CompilerParams(needs_layout_passes=False),
        scratch_types=[
            pltpu.VMEM((B,), jnp.float32),
            pltpu.VMEM((CH, D), jnp.float32),
            pltpu.VMEM((CH, D), jnp.float32),
            pltpu.VMEM((2 * D,), jnp.float32),
            pltpu.VMEM((2 * D,), jnp.float32),
            pltpu.VMEM_SHARED((NS, 2, 2 * D), jnp.float32),
            pltpu.SemaphoreType.DMA,
            pltpu.SemaphoreType.DMA,
        ],
    )(x, current_size)


def kernel(x, current_size):
    return _run(x, current_size)


# speedup vs baseline: 2.1557x; 1.3100x over previous
import functools

import jax
import jax.numpy as jnp
from jax import lax
from jax.experimental import pallas as pl
from jax.experimental.pallas import tpu as pltpu
from jax.experimental.pallas import tpu_sc as plsc

STEP_LEN = 8
MAX_SIZE = 400.0
MIN_SIZE = 1.0
DEFAULT_SIZE = 10.0

NC = 2
NS = 16
L = 16

B = 32
T = 4096
D = 256
ND = D // L
CH = 64
NBUF = 3


def _pool_params(cs_vec):
    ps = jnp.clip(cs_vec + DEFAULT_SIZE, MIN_SIZE, MAX_SIZE)
    trunc_v = ps.astype(jnp.int32)
    frac = ps - trunc_v.astype(jnp.float32)
    nb = (trunc_v + jnp.where(frac > 0.0, 1, 0))[0]
    return ps, trunc_v[0], nb


def _sc_body(x_hbm, cs_hbm, out_hbm, cs_v, buf0, buf1, buf2, acc_v, part_v,
             shared, sem0, sem1, sem2):
    c = lax.axis_index("c")
    s = lax.axis_index("s")
    b1 = 2 * s + c
    b2 = 2 * (NS - 1 - s) + c

    pltpu.sync_copy(cs_hbm, cs_v)

    zero = jnp.zeros((L,), jnp.float32)
    zeros = tuple(zero for _ in range(2 * ND))

    def read_cs(b):
        return plsc.load_gather(cs_v, [jnp.full((L,), b, jnp.int32)])

    bufs = (buf0, buf1, buf2)
    sems = (sem0, sem1, sem2)

    def do_range(b, lo, hi, ps, trunc_s):
        for j in range(2 * ND):
            acc_v[pl.ds(j * L, L)] = zero
        nch = (hi - lo + CH - 1) // CH

        def flush(acc):
            for j in range(2 * ND):
                acc_v[pl.ds(j * L, L)] = acc_v[pl.ds(j * L, L)] + acc[j]

        def process(g, mybuf, mysem, next_buf, next_sem):
            pltpu.make_async_copy(
                x_hbm.at[b, pl.ds(0, CH), :], mybuf, mysem).wait()

            @pl.when(g + NBUF - 1 < nch)
            def _prefetch():
                pltpu.async_copy(
                    x_hbm.at[b, pl.ds(lo + (g + NBUF - 1) * CH, CH), :],
                    next_buf, next_sem)

            start = lo + g * CH
            tb_last = (start + CH - 1) // STEP_LEN
            fast = jnp.logical_and(start + CH <= hi, tb_last + 1 <= trunc_s)

            @pl.when(fast)
            def _fast():
                def row_body(r, acc):
                    new_m = []
                    new_s = []
                    for j in range(ND):
                        v = mybuf[r, pl.ds(j * L, L)]
                        new_m.append(acc[j] + v)
                        new_s.append(acc[ND + j] + v * v)
                    return tuple(new_m + new_s)

                flush(lax.fori_loop(0, CH, row_body, zeros, unroll=2))

            @pl.when(jnp.logical_not(fast))
            def _slow():
                def row_body(r, acc):
                    t = start + r
                    tb = t // STEP_LEN
                    w = jnp.clip(ps - tb.astype(jnp.float32), 0.0, 1.0)
                    w = w * (t < hi).astype(jnp.float32)
                    new_m = []
                    new_s = []
                    for j in range(ND):
                        v = mybuf[r, pl.ds(j * L, L)]
                        wv = w * v
                        new_m.append(acc[j] + wv)
                        new_s.append(acc[ND + j] + wv * v)
                    return tuple(new_m + new_s)

                flush(lax.fori_loop(0, CH, row_body, zeros, unroll=2))

        for k in range(NBUF - 1):
            @pl.when(k < nch)
            def _prime(k=k):
                pltpu.async_copy(
                    x_hbm.at[b, pl.ds(lo + k * CH, CH), :], bufs[k], sems[k])

        def chunk_body(g, carry):
            for k in range(NBUF):
                @pl.when(g % NBUF == k)
                def _proc(k=k):
                    nk = (k + NBUF - 1) % NBUF
                    process(g, bufs[k], sems[k], bufs[nk], sems[nk])

            return carry

        lax.fori_loop(0, nch, chunk_body, 0, unroll=False)

    ps1, tr1, nb1 = _pool_params(read_cs(b1))
    h1 = STEP_LEN * ((nb1 + 1) // 2)
    do_range(b1, 0, h1, ps1, tr1)
    pltpu.sync_copy(acc_v, shared.at[s, 0])

    ps2, tr2, nb2 = _pool_params(read_cs(b2))
    lo2 = STEP_LEN * ((nb2 + 1) // 2)
    hi2 = STEP_LEN * nb2
    do_range(b2, lo2, hi2, ps2, tr2)
    pltpu.sync_copy(acc_v, shared.at[NS - 1 - s, 1])

    plsc.subcore_barrier()

    pltpu.sync_copy(shared.at[s, 0], acc_v)
    pltpu.sync_copy(shared.at[s, 1], part_v)
    inv = 1.0 / (float(STEP_LEN) * ps1)
    for j in range(ND):
        m = (acc_v[pl.ds(j * L, L)] + part_v[pl.ds(j * L, L)]) * inv
        sq = (acc_v[pl.ds((ND + j) * L, L)] + part_v[pl.ds((ND + j) * L, L)]) * inv
        part_v[pl.ds(j * L, L)] = m
        part_v[pl.ds((ND + j) * L, L)] = sq - m * m
    pltpu.sync_copy(part_v, out_hbm.at[b1])


@jax.jit
def _run(x, current_size):
    mesh = plsc.VectorSubcoreMesh(core_axis_name="c", subcore_axis_name="s")
    return pl.kernel(
        _sc_body,
        out_type=jax.ShapeDtypeStruct((B, 2 * D), jnp.float32),
        mesh=mesh,
        compiler_params=pltpu.CompilerParams(needs_layout_passes=False),
        scratch_types=[
            pltpu.VMEM((B,), jnp.float32),
            pltpu.VMEM((CH, D), jnp.float32),
            pltpu.VMEM((CH, D), jnp.float32),
            pltpu.VMEM((CH, D), jnp.float32),
            pltpu.VMEM((2 * D,), jnp.float32),
            pltpu.VMEM((2 * D,), jnp.float32),
            pltpu.VMEM_SHARED((NS, 2, 2 * D), jnp.float32),
            pltpu.SemaphoreType.DMA,
            pltpu.SemaphoreType.DMA,
            pltpu.SemaphoreType.DMA,
        ],
    )(x, current_size)


def kernel(x, current_size):
    return _run(x, current_size)
